# baseline (speedup 1.0000x reference)
import jax
import jax.numpy as jnp
from jax import lax
from jax.experimental import pallas as pl

_DATA_DIM = 26
_DEPTH = 1000
_BATCH = 1024


def _body(xt_ref, o_ref):
    v = lax.broadcasted_iota(jnp.int32, (2 * _DEPTH, _BATCH), 0)
    a = xt_ref[0, 0:1, :]
    b = xt_ref[0, 1:2, :]
    sel = jnp.where(v < _DEPTH, a, b)
    m = v - jnp.where(v >= _DEPTH, _DEPTH, 0)
    o_ref[...] = (m == sel).astype(jnp.float32)


def kernel(x):
    xt = x.T.reshape(_DATA_DIM // 2, 2, _BATCH)
    ot = pl.pallas_call(
        _body,
        grid=(_DATA_DIM // 2,),
        in_specs=[pl.BlockSpec((1, 2, _BATCH), lambda i: (i, 0, 0))],
        out_specs=pl.BlockSpec((2 * _DEPTH, _BATCH), lambda i: (i, 0)),
        out_shape=jax.ShapeDtypeStruct((_DATA_DIM * _DEPTH, _BATCH), jnp.float32),
    )(xt)
    return ot.T


# speedup vs baseline: 1.0335x; 1.0335x over previous
import jax
import jax.numpy as jnp
from jax import lax
from jax.experimental import pallas as pl

_DATA_DIM = 26
_DEPTH = 1000
_BATCH = 1024


def _body(xt_ref, o_ref):
    v = lax.broadcasted_iota(jnp.int32, (_DEPTH, _BATCH), 0)
    o_ref[...] = (v == xt_ref[0]).astype(jnp.float32)


def kernel(x):
    xt = x.T.reshape(_DATA_DIM, 1, _BATCH)
    ot = pl.pallas_call(
        _body,
        grid=(_DATA_DIM,),
        in_specs=[pl.BlockSpec((1, 1, _BATCH), lambda i: (i, 0, 0))],
        out_specs=pl.BlockSpec((_DEPTH, _BATCH), lambda i: (i, 0)),
        out_shape=jax.ShapeDtypeStruct((_DATA_DIM * _DEPTH, _BATCH), jnp.float32),
    )(xt)
    return ot.T
